# Initial kernel scaffold; baseline (speedup 1.0000x reference)
import jax
import jax.numpy as jnp
from jax.experimental import pallas as pl


def kernel(node_feat, edge_feat, edge_index, W1, b1, W2, b2, gamma, beta):
    raise NotImplementedError("write your pallas kernel here")


# speedup vs baseline: 2.4257x; 2.4257x over previous
import jax
import jax.numpy as jnp
from jax import lax
from jax.experimental import pallas as pl
from jax.experimental.pallas import tpu as pltpu
from jax.experimental.pallas import tpu_sc as plsc

N_NODES = 10000
D = 256
DH = 128
D_EDGE = 16
EPS = 1e-5

NC = 2
NS = 16
K = 128
NSTEP = 80
E_TILE = K * NSTEP
E_PAD = E_TILE * NS
N_EDGES = 160000
NSTEP_LAST = (N_EDGES - (NS - 1) * E_TILE) // K
S_ROWS = 10112
STRIPE = S_ROWS // NS
PACK = 16384


def _proj_body(nf_ref, w_ref, o_ref):
    o_ref[0] = jnp.dot(nf_ref[...], w_ref[0],
                       preferred_element_type=jnp.float32)


def _node_proj(node_feat, w1a_t_r):
    grid = 10
    blk = N_NODES // grid
    return pl.pallas_call(
        _proj_body,
        grid=(NC, grid),
        in_specs=[
            pl.BlockSpec((blk, D), lambda c, i: (i, 0)),
            pl.BlockSpec((1, D, DH), lambda c, i: (c, 0, 0)),
        ],
        out_specs=pl.BlockSpec((1, blk, DH), lambda c, i: (c, i, 0)),
        out_shape=jax.ShapeDtypeStruct((NC, N_NODES, DH), jnp.float32),
    )(node_feat, w1a_t_r)


def _eproj_body(ef_ref, w_ref, b_ref, o_ref):
    res = jnp.dot(ef_ref[...], w_ref[0], preferred_element_type=jnp.float32)
    o_ref[0] = res + b_ref[0]


def _edge_proj(edge_feat_p, w1b_t_r, b1_r):
    grid = 80
    blk = E_PAD // grid
    return pl.pallas_call(
        _eproj_body,
        grid=(NC, grid),
        in_specs=[
            pl.BlockSpec((blk, D_EDGE), lambda c, i: (i, 0)),
            pl.BlockSpec((1, D_EDGE, DH), lambda c, i: (c, 0, 0)),
            pl.BlockSpec((1, 1, DH), lambda c, i: (c, 0, 0)),
        ],
        out_specs=pl.BlockSpec((1, blk, DH), lambda c, i: (c, i, 0)),
        out_shape=jax.ShapeDtypeStruct((NC, E_PAD, DH), jnp.float32),
    )(edge_feat_p, w1b_t_r, b1_r)


def _sc_body(proj, eproj, packh, s_out, cnt_out,
             pack_v, srow, drow, grows, erows, ones_v, zc, acc, cnt_acc):
    cid = lax.axis_index("c")
    tid = lax.axis_index("s")

    pltpu.sync_copy(packh.at[tid], pack_v)

    zvec = jnp.zeros((16,), jnp.float32)
    ovec = jnp.ones((16,), jnp.float32)

    def zero_row(r, _):
        for v in range(DH // 16):
            grows[r, pl.ds(v * 16, 16)] = zvec
        return 0

    lax.fori_loop(0, K, zero_row, 0)

    def fill_ones(r, _):
        ones_v[pl.ds(r * 16, 16)] = ovec
        return 0

    lax.fori_loop(0, K // 16, fill_ones, 0)

    def zero_zc(r, _):
        zc[pl.ds(r * 16, 16)] = zvec
        return 0

    lax.fori_loop(0, STRIPE // 16, zero_zc, 0)

    r0 = pl.multiple_of(tid * STRIPE, 8)
    for m in range(STRIPE // K):
        pltpu.sync_copy(grows, acc.at[pl.ds(r0 + m * K, K)])
    rem = STRIPE % K
    if rem:
        pltpu.sync_copy(grows.at[pl.ds(0, rem)],
                        acc.at[pl.ds(r0 + (STRIPE // K) * K, rem)])
    pltpu.sync_copy(zc, cnt_acc.at[pl.ds(r0, STRIPE)])
    plsc.subcore_barrier()

    nsteps = jnp.where(tid == NS - 1, NSTEP_LAST, NSTEP)

    def step(j, _):
        def decode(v, _):
            sl = pl.ds(v * 16, 16)
            p = pack_v[j, sl]
            srow[sl] = p & (PACK - 1)
            drow[sl] = p >> 14
            return 0

        lax.fori_loop(0, K // 16, decode, 0)
        e0 = pl.multiple_of(tid * E_TILE + j * K, K)
        pltpu.sync_copy(proj.at[cid].at[srow], grows)
        pltpu.sync_copy(eproj.at[cid, pl.ds(e0, K)], erows)

        def relu_row(r, _):
            for v in range(DH // 16):
                sl = pl.ds(v * 16, 16)
                grows[r, sl] = jnp.maximum(grows[r, sl] + erows[r, sl], 0.0)
            return 0

        lax.fori_loop(0, K, relu_row, 0)
        pltpu.sync_copy(grows, acc.at[drow], add=True)
        pltpu.sync_copy(ones_v, cnt_acc.at[drow], add=True)
        return 0

    lax.fori_loop(0, nsteps, step, 0)
    plsc.subcore_barrier()

    for m in range(STRIPE // K):
        pltpu.sync_copy(acc.at[pl.ds(r0 + m * K, K)],
                        s_out.at[cid, pl.ds(r0 + m * K, K)])
    if rem:
        pltpu.sync_copy(acc.at[pl.ds(r0 + (STRIPE // K) * K, rem)],
                        s_out.at[cid, pl.ds(r0 + (STRIPE // K) * K, rem)])
    pltpu.sync_copy(cnt_acc.at[pl.ds(r0, STRIPE)], zc)
    pltpu.sync_copy(zc, cnt_out.at[pl.ds(cid * S_ROWS + r0, STRIPE)])


def _sc_aggregate(proj, eproj, pack_r):
    mesh = plsc.VectorSubcoreMesh(core_axis_name="c", subcore_axis_name="s")
    return pl.kernel(
        _sc_body,
        out_type=[
            jax.ShapeDtypeStruct((NC, S_ROWS, DH), jnp.float32),
            jax.ShapeDtypeStruct((NC * S_ROWS,), jnp.float32),
        ],
        mesh=mesh,
        scratch_types=[
            pltpu.VMEM((NSTEP, K), jnp.int32),
            pltpu.VMEM((K,), jnp.int32),
            pltpu.VMEM((K,), jnp.int32),
            pltpu.VMEM((K, DH), jnp.float32),
            pltpu.VMEM((K, DH), jnp.float32),
            pltpu.VMEM((K,), jnp.float32),
            pltpu.VMEM((STRIPE,), jnp.float32),
            pltpu.VMEM_SHARED((S_ROWS, DH), jnp.float32),
            pltpu.VMEM_SHARED((S_ROWS,), jnp.float32),
        ],
    )(proj, eproj, pack_r)


def _final_body(sa_ref, sb_ref, cnt_ref, nf_ref, w2t_ref, b2_ref, g_ref,
                b_ref, o_ref):
    cnt = cnt_ref[...]
    maxc = jnp.maximum(cnt, 1.0)
    s = jnp.concatenate([sa_ref[0], sb_ref[0]], axis=1)
    sbar = s / maxc
    h = jnp.dot(sbar, w2t_ref[...], preferred_element_type=jnp.float32)
    h = h + (cnt / maxc) * b2_ref[...]
    x = h + nf_ref[...]
    mean = jnp.mean(x, axis=-1, keepdims=True)
    xc = x - mean
    var = jnp.mean(xc * xc, axis=-1, keepdims=True)
    o_ref[...] = xc * lax.rsqrt(var + EPS) * g_ref[...] + b_ref[...]


def _final(s, cnt, node_feat, w2t, b2, gamma, beta):
    grid = 10
    blk = N_NODES // grid
    return pl.pallas_call(
        _final_body,
        grid=(grid,),
        in_specs=[
            pl.BlockSpec((1, blk, DH), lambda i: (0, i, 0)),
            pl.BlockSpec((1, blk, DH), lambda i: (1, i, 0)),
            pl.BlockSpec((blk, 1), lambda i: (i, 0)),
            pl.BlockSpec((blk, D), lambda i: (i, 0)),
            pl.BlockSpec((D, D), lambda i: (0, 0)),
            pl.BlockSpec((1, D), lambda i: (0, 0)),
            pl.BlockSpec((1, D), lambda i: (0, 0)),
            pl.BlockSpec((1, D), lambda i: (0, 0)),
        ],
        out_specs=pl.BlockSpec((blk, D), lambda i: (i, 0)),
        out_shape=jax.ShapeDtypeStruct((N_NODES, D), jnp.float32),
    )(s, s, cnt, node_feat, w2t, b2, gamma, beta)


def kernel(node_feat, edge_feat, edge_index, W1, b1, W2, b2, gamma, beta):
    n_edges = edge_index.shape[1]
    w1a_t_r = W1[:, :D].T.reshape(D, NC, DH).transpose(1, 0, 2)
    w1b_t_r = W1[:, D:].T.reshape(D_EDGE, NC, DH).transpose(1, 0, 2)
    b1_r = b1.reshape(NC, 1, DH)
    w2t = W2.T

    src = edge_index[0].astype(jnp.int32)
    dst = edge_index[1].astype(jnp.int32)
    packed = src + dst * PACK
    pad = E_PAD - n_edges
    pack_r = jnp.concatenate([packed, jnp.zeros((pad,), jnp.int32)]).reshape(
        NS, NSTEP, K)
    ef_p = jnp.concatenate(
        [edge_feat, jnp.zeros((pad, D_EDGE), jnp.float32)], axis=0)

    proj = _node_proj(node_feat, w1a_t_r)
    eproj = _edge_proj(ef_p, w1b_t_r, b1_r)
    s, cnt = _sc_aggregate(proj, eproj, pack_r)
    return _final(s, cnt[:N_NODES].reshape(N_NODES, 1), node_feat, w2t,
                  b2.reshape(1, D), gamma.reshape(1, D), beta.reshape(1, D))
